# baseline (speedup 1.0000x reference)
import functools

import jax
import jax.numpy as jnp
from jax import lax
from jax.experimental import pallas as pl
from jax.experimental.pallas import tpu as pltpu
from jax.experimental.pallas import tpu_sc as plsc

N = 10000
E = 320000
G = 256
D = 128
DL = 64
NP_ = 10240
EP = 327680
NW = 32
NSUB = 16
CHUNK = 64
NCH = EP // (NW * CHUNK)
BROWS = NP_ // NW
CNT = 384
ROWT = NP_ // NSUB
RB = 10
TB = NP_ // RB


def _make_agg(with_stats):
    mesh = plsc.VectorSubcoreMesh(core_axis_name="c", subcore_axis_name="s")
    acc_t = jax.ShapeDtypeStruct((2, NP_, D), jnp.float32)
    out_type = [acc_t] if with_stats else acc_t
    scratch = [
        pltpu.VMEM((CHUNK, D), jnp.float32),
        pltpu.VMEM((CHUNK, D), jnp.float32),
        pltpu.VMEM((NCH, CHUNK), jnp.int32),
        pltpu.VMEM((CHUNK,), jnp.int32),
        pltpu.VMEM((CHUNK,), jnp.int32),
        pltpu.VMEM((CHUNK,), jnp.int32),
        pltpu.VMEM((CHUNK,), jnp.int32),
        pltpu.VMEM_SHARED((NP_, D), jnp.float32),
        pltpu.SemaphoreType.DMA,
        pltpu.SemaphoreType.DMA,
    ]
    if with_stats:
        out_type += [
            jax.ShapeDtypeStruct((NW, NP_), jnp.float32),
            jax.ShapeDtypeStruct((NW, CNT), jnp.float32),
        ]
        scratch += [
            pltpu.VMEM((NP_,), jnp.float32),
            pltpu.VMEM((CNT,), jnp.float32),
            pltpu.VMEM((BROWS,), jnp.int32),
        ]

    def body(*refs):
        if with_stats:
            (h_hbm, pkr, batchr, acc_out, degp_out, cntp_out,
             rbuf, rbuf1, pk, sb0, db0, sb1, db1, accsh, sem, sem1,
             degl, cntl, bidx) = refs
        else:
            (h_hbm, pkr, acc_out,
             rbuf, rbuf1, pk, sb0, db0, sb1, db1, accsh, sem, sem1) = refs
        c = lax.axis_index("c")
        s = lax.axis_index("s")
        wid = c * NSUB + s

        z16 = jnp.zeros((16,), jnp.float32)
        one16 = jnp.ones((16,), jnp.float32)
        m16 = jnp.full((16,), 0xFFFF, jnp.int32)

        pkd = pltpu.async_copy(pkr.at[wid], pk, sem)
        if with_stats:
            pltpu.sync_copy(batchr.at[wid], bidx)

        def unpack(j, sb, db):
            for i in range(CHUNK // 16):
                v = pk[j, pl.ds(i * 16, 16)]
                sb[pl.ds(i * 16, 16)] = v & m16
                db[pl.ds(i * 16, 16)] = lax.shift_right_logical(v, 16)

        def zrow(k, _):
            rbuf1[k // 8, pl.ds((k % 8) * 16, 16)] = z16
            return _
        lax.fori_loop(0, CHUNK * D // 16, zrow, None)

        pkd.wait()
        unpack(0, sb0, db0)
        pltpu.async_copy(h_hbm.at[sb0], rbuf, sem)

        def zacc(t, _):
            pltpu.sync_copy(rbuf1,
                            accsh.at[pl.ds(s * ROWT + t * CHUNK, CHUNK)])
            return _
        lax.fori_loop(0, ROWT // CHUNK, zacc, None)

        if with_stats:
            def zdeg(k, _):
                degl[pl.ds(k * 16, 16)] = z16
                return _
            lax.fori_loop(0, NP_ // 16, zdeg, None)

            def zcnt(k, _):
                cntl[pl.ds(k * 16, 16)] = z16
                return _
            lax.fori_loop(0, CNT // 16, zcnt, None)

            def cstep(i, _):
                bv = bidx[pl.ds(i * 16, 16)]
                plsc.addupdate_scatter(cntl, [bv], one16)
                return _
            lax.fori_loop(0, BROWS // 16, cstep, None)

        plsc.subcore_barrier()

        def degs(db):
            if with_stats:
                for i in range(CHUNK // 16):
                    dv = db[pl.ds(i * 16, 16)]
                    plsc.addupdate_scatter(degl, [dv], one16)

        HC = CHUNK // 2

        def gat(sb, rb, sm):
            pltpu.async_copy(h_hbm.at[sb.at[pl.ds(0, HC)]],
                             rb.at[pl.ds(0, HC)], sm)
            pltpu.async_copy(h_hbm.at[sb.at[pl.ds(HC, HC)]],
                             rb.at[pl.ds(HC, HC)], sm)

        def step(j2, _):
            j = 2 * j2
            unpack(j + 1, sb1, db1)
            gat(sb1, rbuf1, sem1)
            pltpu.make_async_copy(h_hbm.at[sb0], rbuf, sem).wait()
            pltpu.sync_copy(rbuf, accsh.at[db0], add=True)
            degs(db0)
            jn = lax.select(j2 == NCH // 2 - 1, 0, j + 2)
            unpack(jn, sb0, db0)
            gat(sb0, rbuf, sem)
            pltpu.make_async_copy(h_hbm.at[sb1], rbuf1, sem1).wait()
            pltpu.sync_copy(rbuf1, accsh.at[db1], add=True)
            degs(db1)
            return _
        lax.fori_loop(0, NCH // 2, step, None)
        pltpu.make_async_copy(h_hbm.at[sb0], rbuf, sem).wait()

        plsc.subcore_barrier()

        sl = pl.ds(s * ROWT, ROWT)
        pltpu.sync_copy(accsh.at[sl], acc_out.at[c, sl])

        if with_stats:
            pltpu.sync_copy(degl, degp_out.at[wid])
            pltpu.sync_copy(cntl, cntp_out.at[wid])

    return pl.kernel(body, out_type=out_type, mesh=mesh,
                     scratch_types=scratch,
                     compiler_params=pltpu.CompilerParams(
                         needs_layout_passes=False))


_agg_stats = _make_agg(True)
_agg_plain = _make_agg(False)


def _sage_body(xb, accb, degpb, ws, wn, b, out):
    deg = jnp.maximum(jnp.sum(degpb[...], axis=0), 1.0)
    acc = accb[0].astype(jnp.float32) + accb[1].astype(jnp.float32)
    agg = acc / deg[:, None]
    h = xb[...].astype(jnp.float32) @ ws[...] + agg @ wn[...] + b[...]
    out[...] = jnp.maximum(h, 0.0).astype(out.dtype)


def _l1(x, acc, degp, w1s, w1n, b1):
    return pl.pallas_call(
        _sage_body,
        grid=(RB,),
        in_specs=[
            pl.BlockSpec((TB, D), lambda i: (i, 0)),
            pl.BlockSpec((2, TB, D), lambda i: (0, i, 0)),
            pl.BlockSpec((NW, TB), lambda i: (0, i)),
            pl.BlockSpec((D, D), lambda i: (0, 0)),
            pl.BlockSpec((D, D), lambda i: (0, 0)),
            pl.BlockSpec((1, D), lambda i: (0, 0)),
        ],
        out_specs=pl.BlockSpec((TB, D), lambda i: (i, 0)),
        out_shape=jax.ShapeDtypeStruct((NP_, D), jnp.float32),
    )(x, acc, degp, w1s, w1n, b1)


def _l2_body(hb, accb, degpb, batchb, ws, wn, b, gout):
    i = pl.program_id(0)
    deg = jnp.maximum(jnp.sum(degpb[...], axis=0), 1.0)
    acc = accb[0].astype(jnp.float32) + accb[1].astype(jnp.float32)
    agg = acc / deg[:, None]
    h2 = jnp.maximum(
        hb[...].astype(jnp.float32) @ ws[...] + agg @ wn[...] + b[...], 0.0)
    bv = batchb[0, 0, :]
    onehot = (bv[:, None] == lax.broadcasted_iota(jnp.int32, (TB, G), 1))
    part = lax.dot_general(onehot.astype(jnp.float32), h2,
                           (((0,), (0,)), ((), ())),
                           preferred_element_type=jnp.float32)

    @pl.when(i == 0)
    def _():
        gout[...] = jnp.zeros_like(gout)

    gout[...] += part


def _l2(h1, acc, degp, batchr, w2s, w2n, b2):
    return pl.pallas_call(
        _l2_body,
        grid=(RB,),
        in_specs=[
            pl.BlockSpec((TB, D), lambda i: (i, 0)),
            pl.BlockSpec((2, TB, D), lambda i: (0, i, 0)),
            pl.BlockSpec((NW, TB), lambda i: (0, i)),
            pl.BlockSpec((1, 1, TB), lambda i: (i, 0, 0)),
            pl.BlockSpec((D, D), lambda i: (0, 0)),
            pl.BlockSpec((D, D), lambda i: (0, 0)),
            pl.BlockSpec((1, D), lambda i: (0, 0)),
        ],
        out_specs=pl.BlockSpec((G, D), lambda i: (0, 0)),
        out_shape=jax.ShapeDtypeStruct((G, D), jnp.float32),
    )(h1, acc, degp, batchr, w2s, w2n, b2)


def _mlp_body(gsum, cntp, wm1, bm1, wm2, bm2, out):
    cnt = jnp.maximum(jnp.sum(cntp[...], axis=0)[:G], 1.0)
    g = gsum[...] / cnt[:, None]
    h = jnp.maximum(g @ wm1[...] + bm1[...], 0.0)
    out[...] = h @ wm2[...] + bm2[...]


def _mlp(gsum, cntp, wm1, bm1, wm2, bm2):
    return pl.pallas_call(
        _mlp_body,
        out_shape=jax.ShapeDtypeStruct((G, DL), jnp.float32),
    )(gsum, cntp, wm1, bm1, wm2, bm2)


def kernel(x, edge_index, batch, W1_self, W1_neigh, b1,
           W2_self, W2_neigh, b2, Wm1, bm1, Wm2, bm2):
    src = edge_index[0].astype(jnp.int32)
    dst = edge_index[1].astype(jnp.int32)
    bat = batch.astype(jnp.int32)

    pad_ar = jnp.arange(EP - E, dtype=jnp.int32)
    src_p = jnp.concatenate([src, pad_ar % N])
    dst_p = jnp.concatenate([dst, N + pad_ar % (NP_ - N)])
    pk_r = ((dst_p << 16) | src_p).reshape(NW, NCH, CHUNK)
    bat_sc = jnp.concatenate(
        [bat, jnp.full((NP_ - N,), G, jnp.int32)]).reshape(NW, BROWS)
    bat_tc = jnp.concatenate(
        [bat, jnp.full((NP_ - N,), G, jnp.int32)]).reshape(RB, 1, TB)
    x_pad = jnp.concatenate([x, jnp.zeros((NP_ - N, D), jnp.float32)])

    b1r = b1.reshape(1, D)
    b2r = b2.reshape(1, D)
    bm1r = bm1.reshape(1, D)
    bm2r = bm2.reshape(1, DL)

    acc1, degp, cntp = _agg_stats(x_pad, pk_r, bat_sc)
    h1 = _l1(x_pad, acc1, degp, W1_self, W1_neigh, b1r)
    acc2 = _agg_plain(h1, pk_r)
    gsum = _l2(h1, acc2, degp, bat_tc, W2_self, W2_neigh, b2r)
    return _mlp(gsum, cntp, Wm1, bm1r, Wm2, bm2r)
